# Initial kernel scaffold; baseline (speedup 1.0000x reference)
import functools

import jax
import jax.numpy as jnp
from jax import lax
from jax.experimental import pallas as pl
from jax.experimental.pallas import tpu as pltpu
from jax.experimental.pallas import tpu_sc as plsc

f32 = jnp.float32
i32 = jnp.int32

N = 100000
P = 100352
E = 1600000
G = 128
K = 30
HID = 32
HALF = 16
NB = P // 1024
RT = 256
NRT = P // RT
TPB = P // 32
EMB_B = 784
EC = 1000
SLOTS = G * K
SLOTS_PAD = 3848
SENT = N
BSENT = 999

_HI = jax.lax.Precision.HIGHEST
_mesh = plsc.VectorSubcoreMesh(core_axis_name="c", subcore_axis_name="s")


def _dot(a, b):
    return jnp.dot(a, b, precision=_HI, preferred_element_type=f32)


@functools.partial(
    pl.kernel,
    out_type=jax.ShapeDtypeStruct((P, HID), f32),
    mesh=_mesh,
    scratch_types=[
        pltpu.VMEM((EMB_B,), i32),
        pltpu.VMEM((EMB_B,), i32),
        pltpu.VMEM((EMB_B,), i32),
        pltpu.VMEM((EMB_B, HID), f32),
        pltpu.VMEM((EMB_B, HID), f32),
        pltpu.VMEM((EMB_B, HID), f32),
        pltpu.SemaphoreType.DMA,
    ],
)
def _sc_emb(x0, x1, dep, t1, a1, d1, z1, i0, i1, i2, bt, ba, bd, sem):
    c = lax.axis_index("c")
    s = lax.axis_index("s")
    wid = s * 2 + c
    nb = wid * TPB
    for b in range(4):
        base = nb + b * EMB_B
        pltpu.sync_copy(x0.at[pl.ds(base, EMB_B)], i0)
        pltpu.sync_copy(x1.at[pl.ds(base, EMB_B)], i1)
        pltpu.sync_copy(dep.at[pl.ds(base, EMB_B)], i2)
        pltpu.async_copy(t1.at[i0], bt, sem).wait()
        pltpu.async_copy(a1.at[i1], ba, sem).wait()
        pltpu.async_copy(d1.at[i2], bd, sem).wait()

        def add(e, carry):
            for h in range(2):
                sl = pl.ds(h * HALF, HALF)
                bt[e, sl] = bt[e, sl] + ba[e, sl] + bd[e, sl]
            return carry

        lax.fori_loop(0, EMB_B, add, 0)
        pltpu.sync_copy(bt, z1.at[pl.ds(base, EMB_B)])


@functools.partial(
    pl.kernel,
    out_type=jax.ShapeDtypeStruct((2, P, HALF), f32),
    mesh=_mesh,
    scratch_types=[
        pltpu.VMEM((EC,), i32),
        pltpu.VMEM((EC, HALF), f32),
        pltpu.VMEM_SHARED((P, HALF), f32),
    ],
)
def _sc_deg(dst, zeros16, erow, deg_out, dbuf, obuf, acc):
    c = lax.axis_index("c")
    s = lax.axis_index("s")
    pltpu.sync_copy(zeros16.at[pl.ds(s * TPB, TPB)], acc.at[pl.ds(s * TPB, TPB)])
    pltpu.sync_copy(erow, obuf)
    plsc.subcore_barrier()
    ebase = c * (E // 2) + s * (E // 32)

    def chunk(idx, carry):
        e0 = ebase + idx * EC
        pltpu.sync_copy(dst.at[pl.ds(e0, EC)], dbuf)
        pltpu.sync_copy(obuf, acc.at[dbuf], add=True)
        return carry

    lax.fori_loop(0, (E // 32) // EC, chunk, 0)
    plsc.subcore_barrier()
    pltpu.sync_copy(acc.at[pl.ds(s * TPB, TPB)], deg_out.at[c, pl.ds(s * TPB, TPB)])


@functools.partial(
    pl.kernel,
    out_type=jax.ShapeDtypeStruct((2, P, HALF), f32),
    mesh=_mesh,
    scratch_types=[
        pltpu.VMEM((EC,), i32),
        pltpu.VMEM((EC,), i32),
        pltpu.VMEM((EC, HALF), f32),
        pltpu.VMEM_SHARED((P, HALF), f32),
        pltpu.SemaphoreType.DMA,
    ],
)
def _sc_agg(src, dst, zs2, zeros16, acc_out, sbuf, dbuf, rows, acc, sem):
    c = lax.axis_index("c")
    s = lax.axis_index("s")
    pltpu.sync_copy(zeros16.at[pl.ds(s * TPB, TPB)], acc.at[pl.ds(s * TPB, TPB)])
    plsc.subcore_barrier()
    ebase = s * (E // 16)

    def chunk(idx, carry):
        e0 = ebase + idx * EC
        pltpu.sync_copy(src.at[pl.ds(e0, EC)], sbuf)
        pltpu.sync_copy(dst.at[pl.ds(e0, EC)], dbuf)
        pltpu.async_copy(zs2.at[c].at[sbuf], rows, sem).wait()
        pltpu.sync_copy(rows, acc.at[dbuf], add=True)
        return carry

    lax.fori_loop(0, (E // 16) // EC, chunk, 0)
    plsc.subcore_barrier()
    pltpu.sync_copy(acc.at[pl.ds(s * TPB, TPB)], acc_out.at[c, pl.ds(s * TPB, TPB)])


@functools.partial(
    pl.kernel,
    out_type=jax.ShapeDtypeStruct((SLOTS,), i32),
    mesh=_mesh,
    scratch_types=[
        pltpu.VMEM((1024,), i32),
        pltpu.VMEM((SLOTS_PAD,), i32),
    ],
)
def _sc_slots(dest, init, slots_out, dbuf, slots):
    c = lax.axis_index("c")
    s = lax.axis_index("s")

    @pl.when(jnp.logical_and(c == 0, s == 0))
    def _():
        pltpu.sync_copy(init, slots)

        def blk(b, carry):
            pltpu.sync_copy(dest.at[pl.ds(b * 1024, 1024)], dbuf)

            def inner(k, c2):
                iv = dbuf[pl.ds(k * 16, 16)]
                vals = lax.iota(i32, 16) + (b * 1024 + k * 16)
                plsc.store_scatter(slots, [iv], vals)
                return c2

            lax.fori_loop(0, 64, inner, 0)
            return carry

        lax.fori_loop(0, P // 1024, blk, 0)
        pltpu.sync_copy(slots.at[pl.ds(0, SLOTS)], slots_out)


@functools.partial(
    pl.kernel,
    out_type=[
        jax.ShapeDtypeStruct((SLOTS, HID), f32),
        jax.ShapeDtypeStruct((SLOTS, HID), f32),
        jax.ShapeDtypeStruct((SLOTS, HID), f32),
        jax.ShapeDtypeStruct((SLOTS, HALF), f32),
    ],
    mesh=_mesh,
    scratch_types=[
        pltpu.VMEM((SLOTS // 32,), i32),
        pltpu.VMEM((SLOTS // 32, HID), f32),
        pltpu.VMEM((SLOTS // 32, HALF), f32),
        pltpu.SemaphoreType.DMA,
    ],
)
def _sc_pool(slots, h1, h2, h3, h4p, p1, p2, p3, p4, ibuf, b32, b16, sem):
    c = lax.axis_index("c")
    s = lax.axis_index("s")
    wid = s * 2 + c
    nb = SLOTS // 32
    base = wid * nb
    pltpu.sync_copy(slots.at[pl.ds(base, nb)], ibuf)
    for href, pref in ((h1, p1), (h2, p2), (h3, p3)):
        pltpu.async_copy(href.at[ibuf], b32, sem).wait()
        pltpu.sync_copy(b32, pref.at[pl.ds(base, nb)])
    pltpu.async_copy(h4p.at[ibuf], b16, sem).wait()
    pltpu.sync_copy(b16, p4.at[pl.ds(base, nb)])


def _premul_body(tt, at_, dt, w1, o1, o2, o3):
    o1[...] = _dot(tt[...], w1[...])
    o2[...] = _dot(at_[...], w1[...])
    o3[...] = _dot(dt[...], w1[...])


def _tc_premul(type_table, attr_table, depth_table, w1):
    nt, na, nd = type_table.shape[0], attr_table.shape[0], depth_table.shape[0]
    return pl.pallas_call(
        _premul_body,
        out_shape=[
            jax.ShapeDtypeStruct((nt, HID), f32),
            jax.ShapeDtypeStruct((na, HID), f32),
            jax.ShapeDtypeStruct((nd, HID), f32),
        ],
    )(type_table, attr_table, depth_table, w1)


def _prep1_body(deg2, z1, dinv, zs2):
    d = deg2[0, :, 0:1] + deg2[1, :, 0:1] + 1.0
    di = lax.rsqrt(d)
    zs = di * z1[...]
    dinv[...] = di
    zs2[...] = jnp.stack([zs[:, 0:HALF], zs[:, HALF:HID]], axis=0)


def _tc_prep1(deg2, z1):
    return pl.pallas_call(
        _prep1_body,
        grid=(NB,),
        in_specs=[
            pl.BlockSpec((2, 1024, HALF), lambda i: (0, i, 0)),
            pl.BlockSpec((1024, HID), lambda i: (i, 0)),
        ],
        out_specs=[
            pl.BlockSpec((1024, 1), lambda i: (i, 0)),
            pl.BlockSpec((2, 1024, HALF), lambda i: (0, i, 0)),
        ],
        out_shape=[
            jax.ShapeDtypeStruct((P, 1), f32),
            jax.ShapeDtypeStruct((2, P, HALF), f32),
        ],
    )(deg2, z1)


def _epi_body(acc2, zs2, dinv, bvec, wnext, h_out, zsn2, *, last):
    pid = pl.program_id(0)
    di = dinv[...]
    z = jnp.concatenate([acc2[0] + zs2[0], acc2[1] + zs2[1]], axis=1)
    h = jnp.tanh(di * z + bvec[...])
    rid = pid * 1024 + lax.broadcasted_iota(i32, (1024, 1), 0)
    h = jnp.where(rid < N, h, 0.0)
    h_out[...] = h
    zs = di * _dot(h, wnext[...])
    if last:
        onehot = (lax.broadcasted_iota(i32, (1, HALF), 1) == 0).astype(f32)
        zs = zs * onehot
        zsn2[...] = jnp.stack([zs, zs], axis=0)
    else:
        zsn2[...] = jnp.stack([zs[:, 0:HALF], zs[:, HALF:HID]], axis=0)


def _tc_epi(acc2, zs2, dinv, bvec, wnext, last=False):
    wn = wnext.shape[1]
    return pl.pallas_call(
        functools.partial(_epi_body, last=last),
        grid=(NB,),
        in_specs=[
            pl.BlockSpec((2, 1024, HALF), lambda i: (0, i, 0)),
            pl.BlockSpec((2, 1024, HALF), lambda i: (0, i, 0)),
            pl.BlockSpec((1024, 1), lambda i: (i, 0)),
            pl.BlockSpec((1, HID), lambda i: (0, 0)),
            pl.BlockSpec((HID, wn), lambda i: (0, 0)),
        ],
        out_specs=[
            pl.BlockSpec((1024, HID), lambda i: (i, 0)),
            pl.BlockSpec((2, 1024, HALF), lambda i: (0, i, 0)),
        ],
        out_shape=[
            jax.ShapeDtypeStruct((P, HID), f32),
            jax.ShapeDtypeStruct((2, P, HALF), f32),
        ],
    )(acc2, zs2, dinv, bvec, wnext)


def _epi4_body(acc2, zs2, dinv, bvec, h4p):
    pid = pl.program_id(0)
    a = acc2[0, :, 0:1] + zs2[0, :, 0:1]
    h = jnp.tanh(dinv[...] * a + bvec[...])
    rid = pid * 1024 + lax.broadcasted_iota(i32, (1024, 1), 0)
    h = jnp.where(rid < N, h, 0.0)
    onehot = (lax.broadcasted_iota(i32, (1, HALF), 1) == 0).astype(f32)
    h4p[...] = h * onehot


def _tc_epi4(acc2, zs2, dinv, bvec):
    return pl.pallas_call(
        _epi4_body,
        grid=(NB,),
        in_specs=[
            pl.BlockSpec((2, 1024, HALF), lambda i: (0, i, 0)),
            pl.BlockSpec((2, 1024, HALF), lambda i: (0, i, 0)),
            pl.BlockSpec((1024, 1), lambda i: (i, 0)),
            pl.BlockSpec((1, 1), lambda i: (0, 0)),
        ],
        out_specs=pl.BlockSpec((1024, HALF), lambda i: (i, 0)),
        out_shape=jax.ShapeDtypeStruct((P, HALF), f32),
    )(acc2, zs2, dinv, bvec)


def _ends_body(bb, st, en):
    pid = pl.program_id(0)

    @pl.when(pid == 0)
    def _():
        st[...] = jnp.zeros_like(st[...])
        en[...] = jnp.zeros_like(en[...])

    bmat = jnp.transpose(jnp.broadcast_to(bb[...], (RT, RT)))[:, 0:G]
    g = lax.broadcasted_iota(f32, (RT, G), 1)
    st[...] += jnp.sum((bmat < g).astype(f32), axis=0, keepdims=True)
    en[...] += jnp.sum((bmat <= g).astype(f32), axis=0, keepdims=True)


def _tc_ends(batchf2):
    return pl.pallas_call(
        _ends_body,
        grid=(NRT,),
        in_specs=[pl.BlockSpec((1, RT), lambda i: (i, 0))],
        out_specs=[
            pl.BlockSpec((1, G), lambda i: (0, 0)),
            pl.BlockSpec((1, G), lambda i: (0, 0)),
        ],
        out_shape=[
            jax.ShapeDtypeStruct((1, G), f32),
            jax.ShapeDtypeStruct((1, G), f32),
        ],
    )(batchf2)


def _rank_body(se, b01, keys, bat, dest):
    t = pl.program_id(0)
    b0 = b01[t, 0]
    b1 = b01[t, 1]
    jlo = se[b0, 0]
    jhi = se[b1, 1]
    c0 = jlo // RT
    c1 = (jhi + RT - 1) // RT
    ki = jnp.broadcast_to(keys[pl.ds(t, 1), :], (RT, RT))
    bi = jnp.broadcast_to(bat[pl.ds(t, 1), :], (RT, RT))
    ii = lax.broadcasted_iota(i32, (RT, RT), 1) + t * RT

    def chunk(cc, cnt):
        kj = jnp.transpose(jnp.broadcast_to(keys[pl.ds(cc, 1), :], (RT, RT)))
        bj = jnp.transpose(jnp.broadcast_to(bat[pl.ds(cc, 1), :], (RT, RT)))
        jj = lax.broadcasted_iota(i32, (RT, RT), 0) + cc * RT
        better = (kj > ki) | ((kj == ki) & (jj < ii))
        m = (bj == bi) & better
        return cnt + jnp.sum(m.astype(f32), axis=0)

    cnt = lax.fori_loop(c0, c1, chunk, jnp.zeros((RT,), f32))
    rank = cnt.astype(i32).reshape(1, RT)
    bi_i = bat[pl.ds(t, 1), :].astype(i32)
    valid = (bi_i < G) & (rank < K)
    dest[...] = jnp.where(valid, bi_i * K + rank, SLOTS)


def _tc_rank(se, b01, keys2, batchf2):
    return pl.pallas_call(
        _rank_body,
        grid=(NRT,),
        in_specs=[
            pl.BlockSpec(memory_space=pltpu.SMEM),
            pl.BlockSpec(memory_space=pltpu.SMEM),
            pl.BlockSpec((NRT, RT), lambda i: (0, 0)),
            pl.BlockSpec((NRT, RT), lambda i: (0, 0)),
        ],
        out_specs=pl.BlockSpec((1, RT), lambda i: (i, 0)),
        out_shape=jax.ShapeDtypeStruct((NRT, RT), i32),
    )(se, b01, keys2, batchf2)


def _head1_body(p1, p2, p3, p4, w97, c1b, w2, c2b, cflat):
    wa = w97[0:32, :]
    wb = w97[32:64, :]
    wc = w97[64:96, :]
    wd = w97[96:97, :]
    c1 = _dot(p1[...], wa) + _dot(p2[...], wb) + _dot(p3[...], wc)
    c1 = c1 + p4[..., 0:1] * wd + c1b[...]
    c1 = jnp.maximum(c1, 0.0)
    r = c1.reshape(G, K, 16).reshape(G, K // 2, 2, 16).max(axis=2)
    acc = jnp.zeros((G, 11, 32), f32)
    for w in range(5):
        acc = acc + lax.dot_general(
            r[:, w:w + 11, :], w2[w], (((2,), (0,)), ((), ())),
            precision=_HI, preferred_element_type=f32)
    c2 = jnp.maximum(acc + c2b[...], 0.0)
    cflat[...] = c2.reshape(G, 352)


def _tc_head1(p1, p2, p3, p4, w97, c1b, w2, c2b):
    return pl.pallas_call(
        _head1_body,
        out_shape=jax.ShapeDtypeStruct((G, 352), f32),
    )(p1, p2, p3, p4, w97, c1b, w2, c2b)


def _head2_body(cf, w, b, o):
    o[...] = (_dot(cf[...], w[0]) + b[...])[None]


def _tc_head2(cflat, linwr, linb):
    nv = linwr.shape[2]
    vb = 1000
    return pl.pallas_call(
        _head2_body,
        grid=(5, nv // vb),
        in_specs=[
            pl.BlockSpec((G, 352), lambda i, v: (0, 0)),
            pl.BlockSpec((1, 352, vb), lambda i, v: (i, 0, v)),
            pl.BlockSpec((1, vb), lambda i, v: (i, v)),
        ],
        out_specs=pl.BlockSpec((1, G, vb), lambda i, v: (i, 0, v)),
        out_shape=jax.ShapeDtypeStruct((5, G, nv), f32),
    )(cflat, linwr, linb)


def kernel(x, edge_index, node_depth, batch, type_table, attr_table,
           depth_table, W1, b1, W2, b2, W3, b3, W4, b4,
           conv1w, conv1b, conv2w, conv2b, linW, linb):
    x0 = jnp.pad(x[:, 0].astype(i32), (0, P - N))
    x1 = jnp.pad(x[:, 1].astype(i32), (0, P - N))
    dep = jnp.pad(node_depth[:, 0].astype(i32), (0, P - N))
    src = edge_index[0].astype(i32)
    dst = edge_index[1].astype(i32)
    batch_p = jnp.pad(batch.astype(i32), (0, P - N), constant_values=BSENT)
    batchf2 = batch_p.astype(f32).reshape(NRT, RT)
    b01 = jnp.stack([batch_p.reshape(NRT, RT)[:, 0],
                     batch_p.reshape(NRT, RT)[:, RT - 1]], axis=1)
    zeros16 = jnp.zeros((P, HALF), f32)
    erow = jnp.zeros((EC, HALF), f32).at[:, 0].set(1.0)
    slots_init = jnp.full((SLOTS_PAD,), SENT, i32)
    b1r = b1.reshape(1, HID)
    b2r = b2.reshape(1, HID)
    b3r = b3.reshape(1, HID)
    b4r = b4.reshape(1, 1)
    w97 = jnp.transpose(conv1w[:, 0, :])
    c1br = conv1b.reshape(1, 16)
    w2t = jnp.transpose(conv2w, (2, 1, 0))
    c2br = conv2b.reshape(1, 1, 32)
    linwr = jnp.transpose(linW.reshape(5, 32, 11, 5000),
                          (0, 2, 1, 3)).reshape(5, 352, 5000)

    t1, a1, d1 = _tc_premul(type_table, attr_table, depth_table, W1)
    z1 = _sc_emb(x0, x1, dep, t1, a1, d1)
    deg2 = _sc_deg(dst, zeros16, erow)
    dinv, zs = _tc_prep1(deg2, z1)

    acc = _sc_agg(src, dst, zs, zeros16)
    h1, zs = _tc_epi(acc, zs, dinv, b1r, W2)
    acc = _sc_agg(src, dst, zs, zeros16)
    h2, zs = _tc_epi(acc, zs, dinv, b2r, W3)
    acc = _sc_agg(src, dst, zs, zeros16)
    h3, zs = _tc_epi(acc, zs, dinv, b3r, W4, last=True)
    acc = _sc_agg(src, dst, zs, zeros16)
    h4p = _tc_epi4(acc, zs, dinv, b4r)

    keys2 = h4p[:, 0].reshape(NRT, RT)
    st, en = _tc_ends(batchf2)
    se = jnp.concatenate([st.reshape(G, 1), en.reshape(G, 1)],
                         axis=1).astype(i32)
    dest2 = _tc_rank(se, b01, keys2, batchf2)
    slots = _sc_slots(dest2.reshape(P), slots_init)
    p1, p2, p3, p4 = _sc_pool(slots, h1, h2, h3, h4p)

    cflat = _tc_head1(p1, p2, p3, p4, w97, c1br, w2t, c2br)
    out = _tc_head2(cflat, linwr, linb)
    return tuple(out[idx] for idx in range(5))


# speedup vs baseline: 15.7088x; 15.7088x over previous
import functools

import jax
import jax.numpy as jnp
from jax import lax
from jax.experimental import pallas as pl
from jax.experimental.pallas import tpu as pltpu
from jax.experimental.pallas import tpu_sc as plsc

f32 = jnp.float32
i32 = jnp.int32

N = 100000
P = 100352
E = 1600000
G = 128
K = 30
HID = 32
HALF = 16
NB = P // 1024
RT = 256
NRT = P // RT
TPB = P // 32
EMB_B = 784
EC = 1000
SLOTS = G * K
SLOTS_PAD = 3848
SENT = N
BSENT = 999

_HI = jax.lax.Precision.HIGHEST
_mesh = plsc.VectorSubcoreMesh(core_axis_name="c", subcore_axis_name="s")


def _dot(a, b):
    return jnp.dot(a, b, precision=_HI, preferred_element_type=f32)


@functools.partial(
    pl.kernel,
    out_type=jax.ShapeDtypeStruct((P, HID), f32),
    mesh=_mesh,
    compiler_params=pltpu.CompilerParams(use_tc_tiling_on_sc=False),
    scratch_types=[
        pltpu.VMEM((EMB_B,), i32),
        pltpu.VMEM((EMB_B,), i32),
        pltpu.VMEM((EMB_B,), i32),
        pltpu.VMEM((EMB_B, HID), f32),
        pltpu.VMEM((EMB_B, HID), f32),
        pltpu.VMEM((EMB_B, HID), f32),
        pltpu.SemaphoreType.DMA,
    ],
)
def _sc_emb(x0, x1, dep, t1, a1, d1, z1, i0, i1, i2, bt, ba, bd, sem):
    c = lax.axis_index("c")
    s = lax.axis_index("s")
    wid = s * 2 + c
    nb = wid * TPB
    for b in range(4):
        base = nb + b * EMB_B
        pltpu.sync_copy(x0.at[pl.ds(base, EMB_B)], i0)
        pltpu.sync_copy(x1.at[pl.ds(base, EMB_B)], i1)
        pltpu.sync_copy(dep.at[pl.ds(base, EMB_B)], i2)
        pltpu.async_copy(t1.at[i0], bt, sem).wait()
        pltpu.async_copy(a1.at[i1], ba, sem).wait()
        pltpu.async_copy(d1.at[i2], bd, sem).wait()

        def add(e, carry):
            for h in range(2):
                sl = pl.ds(h * HALF, HALF)
                bt[e, sl] = bt[e, sl] + ba[e, sl] + bd[e, sl]
            return carry

        lax.fori_loop(0, EMB_B, add, 0)
        pltpu.sync_copy(bt, z1.at[pl.ds(base, EMB_B)])


@functools.partial(
    pl.kernel,
    out_type=jax.ShapeDtypeStruct((2, P, HALF), f32),
    mesh=_mesh,
    compiler_params=pltpu.CompilerParams(use_tc_tiling_on_sc=False),
    scratch_types=[
        pltpu.VMEM((EC,), i32),
        pltpu.VMEM((EC,), i32),
        pltpu.VMEM((EC, HALF), f32),
        pltpu.VMEM_SHARED((P, HALF), f32),
        pltpu.SemaphoreType.DMA,
    ],
)
def _sc_agg(src, dst, zs2, zeros16, acc_out, sbuf, dbuf, rows, acc, sem):
    c = lax.axis_index("c")
    s = lax.axis_index("s")
    pltpu.sync_copy(zeros16.at[pl.ds(s * TPB, TPB)], acc.at[pl.ds(s * TPB, TPB)])
    plsc.subcore_barrier()
    ebase = s * (E // 16)

    def chunk(idx, carry):
        e0 = ebase + idx * EC
        pltpu.sync_copy(src.at[pl.ds(e0, EC)], sbuf)
        pltpu.sync_copy(dst.at[pl.ds(e0, EC)], dbuf)
        pltpu.async_copy(zs2.at[c].at[sbuf], rows, sem).wait()
        pltpu.sync_copy(rows, acc.at[dbuf], add=True)
        return carry

    lax.fori_loop(0, (E // 16) // EC, chunk, 0)
    plsc.subcore_barrier()
    pltpu.sync_copy(acc.at[pl.ds(s * TPB, TPB)], acc_out.at[c, pl.ds(s * TPB, TPB)])


@functools.partial(
    pl.kernel,
    out_type=jax.ShapeDtypeStruct((SLOTS,), i32),
    mesh=_mesh,
    compiler_params=pltpu.CompilerParams(use_tc_tiling_on_sc=False,
                                         needs_layout_passes=False),
    scratch_types=[
        pltpu.VMEM((1024,), i32),
        pltpu.VMEM((SLOTS_PAD,), i32),
    ],
)
def _sc_slots(dest, init, slots_out, dbuf, slots):
    c = lax.axis_index("c")
    s = lax.axis_index("s")

    @pl.when(jnp.logical_and(c == 0, s == 0))
    def _():
        pltpu.sync_copy(init, slots)

        def blk(b, carry):
            pltpu.sync_copy(dest.at[pl.ds(b * 1024, 1024)], dbuf)

            def inner(k, c2):
                iv = dbuf[pl.ds(k * 16, 16)]
                vals = lax.iota(i32, 16) + (b * 1024 + k * 16)
                plsc.store_scatter(slots, [iv], vals)
                return c2

            lax.fori_loop(0, 64, inner, 0)
            return carry

        lax.fori_loop(0, P // 1024, blk, 0)
        pltpu.sync_copy(slots.at[pl.ds(0, SLOTS)], slots_out)


@functools.partial(
    pl.kernel,
    out_type=[
        jax.ShapeDtypeStruct((SLOTS, HID), f32),
        jax.ShapeDtypeStruct((SLOTS, HID), f32),
        jax.ShapeDtypeStruct((SLOTS, HID), f32),
        jax.ShapeDtypeStruct((SLOTS, HALF), f32),
    ],
    mesh=_mesh,
    compiler_params=pltpu.CompilerParams(use_tc_tiling_on_sc=False),
    scratch_types=[
        pltpu.VMEM((SLOTS // 32,), i32),
        pltpu.VMEM((SLOTS // 32, HID), f32),
        pltpu.VMEM((SLOTS // 32, HALF), f32),
        pltpu.SemaphoreType.DMA,
    ],
)
def _sc_pool(slots, h1, h2, h3, h4p, p1, p2, p3, p4, ibuf, b32, b16, sem):
    c = lax.axis_index("c")
    s = lax.axis_index("s")
    wid = s * 2 + c
    nb = SLOTS // 32
    base = wid * nb
    pltpu.sync_copy(slots.at[pl.ds(base, nb)], ibuf)
    for href, pref in ((h1, p1), (h2, p2), (h3, p3)):
        pltpu.async_copy(href.at[ibuf], b32, sem).wait()
        pltpu.sync_copy(b32, pref.at[pl.ds(base, nb)])
    pltpu.async_copy(h4p.at[ibuf], b16, sem).wait()
    pltpu.sync_copy(b16, p4.at[pl.ds(base, nb)])


def _premul_body(tt, at_, dt, w1, o1, o2, o3):
    o1[...] = _dot(tt[...], w1[...])
    o2[...] = _dot(at_[...], w1[...])
    o3[...] = _dot(dt[...], w1[...])


def _tc_premul(type_table, attr_table, depth_table, w1):
    nt, na, nd = type_table.shape[0], attr_table.shape[0], depth_table.shape[0]
    return pl.pallas_call(
        _premul_body,
        out_shape=[
            jax.ShapeDtypeStruct((nt, HID), f32),
            jax.ShapeDtypeStruct((na, HID), f32),
            jax.ShapeDtypeStruct((nd, HID), f32),
        ],
    )(type_table, attr_table, depth_table, w1)


def _prep1_body(deg2, z1, dinv, zs2):
    d = deg2[0, :, 0:1] + deg2[1, :, 0:1] + 1.0
    di = lax.rsqrt(d)
    zs = di * z1[...]
    dinv[...] = di
    zs2[...] = jnp.stack([zs[:, 0:HALF], zs[:, HALF:HID]], axis=0)


def _tc_prep1(deg2, z1):
    return pl.pallas_call(
        _prep1_body,
        grid=(NB,),
        in_specs=[
            pl.BlockSpec((2, 1024, HALF), lambda i: (0, i, 0)),
            pl.BlockSpec((1024, HID), lambda i: (i, 0)),
        ],
        out_specs=[
            pl.BlockSpec((1024, 1), lambda i: (i, 0)),
            pl.BlockSpec((2, 1024, HALF), lambda i: (0, i, 0)),
        ],
        out_shape=[
            jax.ShapeDtypeStruct((P, 1), f32),
            jax.ShapeDtypeStruct((2, P, HALF), f32),
        ],
    )(deg2, z1)


def _epi_body(acc2, zs2, dinv, bvec, wnext, h_out, zsn2, *, last):
    pid = pl.program_id(0)
    di = dinv[...]
    z = jnp.concatenate([acc2[0] + zs2[0], acc2[1] + zs2[1]], axis=1)
    h = jnp.tanh(di * z + bvec[...])
    rid = pid * 1024 + lax.broadcasted_iota(i32, (1024, 1), 0)
    h = jnp.where(rid < N, h, 0.0)
    h_out[...] = h
    zs = di * _dot(h, wnext[...])
    if last:
        onehot = (lax.broadcasted_iota(i32, (1, HALF), 1) == 0).astype(f32)
        zs = zs * onehot
        zsn2[...] = jnp.stack([zs, zs], axis=0)
    else:
        zsn2[...] = jnp.stack([zs[:, 0:HALF], zs[:, HALF:HID]], axis=0)


def _tc_epi(acc2, zs2, dinv, bvec, wnext, last=False):
    wn = wnext.shape[1]
    return pl.pallas_call(
        functools.partial(_epi_body, last=last),
        grid=(NB,),
        in_specs=[
            pl.BlockSpec((2, 1024, HALF), lambda i: (0, i, 0)),
            pl.BlockSpec((2, 1024, HALF), lambda i: (0, i, 0)),
            pl.BlockSpec((1024, 1), lambda i: (i, 0)),
            pl.BlockSpec((1, HID), lambda i: (0, 0)),
            pl.BlockSpec((HID, wn), lambda i: (0, 0)),
        ],
        out_specs=[
            pl.BlockSpec((1024, HID), lambda i: (i, 0)),
            pl.BlockSpec((2, 1024, HALF), lambda i: (0, i, 0)),
        ],
        out_shape=[
            jax.ShapeDtypeStruct((P, HID), f32),
            jax.ShapeDtypeStruct((2, P, HALF), f32),
        ],
    )(acc2, zs2, dinv, bvec, wnext)


def _epi4_body(acc2, zs2, dinv, bvec, h4p):
    pid = pl.program_id(0)
    a = acc2[0, :, 0:1] + zs2[0, :, 0:1]
    h = jnp.tanh(dinv[...] * a + bvec[...])
    rid = pid * 1024 + lax.broadcasted_iota(i32, (1024, 1), 0)
    h = jnp.where(rid < N, h, 0.0)
    onehot = (lax.broadcasted_iota(i32, (1, HALF), 1) == 0).astype(f32)
    h4p[...] = h * onehot


def _tc_epi4(acc2, zs2, dinv, bvec):
    return pl.pallas_call(
        _epi4_body,
        grid=(NB,),
        in_specs=[
            pl.BlockSpec((2, 1024, HALF), lambda i: (0, i, 0)),
            pl.BlockSpec((2, 1024, HALF), lambda i: (0, i, 0)),
            pl.BlockSpec((1024, 1), lambda i: (i, 0)),
            pl.BlockSpec((1, 1), lambda i: (0, 0)),
        ],
        out_specs=pl.BlockSpec((1024, HALF), lambda i: (i, 0)),
        out_shape=jax.ShapeDtypeStruct((P, HALF), f32),
    )(acc2, zs2, dinv, bvec)


def _ends_body(bb, st, en):
    pid = pl.program_id(0)

    @pl.when(pid == 0)
    def _():
        st[...] = jnp.zeros_like(st[...])
        en[...] = jnp.zeros_like(en[...])

    bmat = jnp.transpose(jnp.broadcast_to(bb[0], (RT, RT)))[:, 0:G]
    g = lax.broadcasted_iota(i32, (RT, G), 1).astype(f32)
    st[...] += jnp.sum((bmat < g).astype(f32), axis=0, keepdims=True)
    en[...] += jnp.sum((bmat <= g).astype(f32), axis=0, keepdims=True)


def _tc_ends(batchf2):
    return pl.pallas_call(
        _ends_body,
        grid=(NRT,),
        in_specs=[pl.BlockSpec((1, 1, RT), lambda i: (i, 0, 0))],
        out_specs=[
            pl.BlockSpec((1, G), lambda i: (0, 0)),
            pl.BlockSpec((1, G), lambda i: (0, 0)),
        ],
        out_shape=[
            jax.ShapeDtypeStruct((1, G), f32),
            jax.ShapeDtypeStruct((1, G), f32),
        ],
    )(batchf2.reshape(NRT, 1, RT))


def _rank_body(se, b01, keys, bat, dest):
    t = pl.program_id(0)
    b0 = b01[t, 0]
    b1 = b01[t, 1]
    jlo = se[b0, 0]
    jhi = se[b1, 1]
    c0 = jlo // RT
    c1 = (jhi + RT - 1) // RT
    ki = jnp.broadcast_to(keys[pl.ds(t, 1), :], (RT, RT))
    bi = jnp.broadcast_to(bat[pl.ds(t, 1), :], (RT, RT))
    ii = lax.broadcasted_iota(i32, (RT, RT), 1) + t * RT

    def chunk(cc, cnt):
        kj = jnp.transpose(jnp.broadcast_to(keys[pl.ds(cc, 1), :], (RT, RT)))
        bj = jnp.transpose(jnp.broadcast_to(bat[pl.ds(cc, 1), :], (RT, RT)))
        jj = lax.broadcasted_iota(i32, (RT, RT), 0) + cc * RT
        better = (kj > ki) | ((kj == ki) & (jj < ii))
        m = (bj == bi) & better
        return cnt + jnp.sum(m.astype(f32), axis=0)

    cnt = lax.fori_loop(c0, c1, chunk, jnp.zeros((RT,), f32))
    rank = cnt.astype(i32).reshape(1, RT)
    bi_i = bat[pl.ds(t, 1), :].astype(i32)
    valid = (bi_i < G) & (rank < K)
    dest[...] = jnp.where(valid, bi_i * K + rank, SLOTS)[None]


def _tc_rank(se, b01, keys2, batchf2):
    return pl.pallas_call(
        _rank_body,
        grid=(NRT,),
        in_specs=[
            pl.BlockSpec(memory_space=pltpu.SMEM),
            pl.BlockSpec(memory_space=pltpu.SMEM),
            pl.BlockSpec((NRT, RT), lambda i: (0, 0)),
            pl.BlockSpec((NRT, RT), lambda i: (0, 0)),
        ],
        out_specs=pl.BlockSpec((1, 1, RT), lambda i: (i, 0, 0)),
        out_shape=jax.ShapeDtypeStruct((NRT, 1, RT), i32),
    )(se, b01, keys2, batchf2)


def _head1_body(p1, p2, p3, p4, w97, c1b, w2, c2b, cflat):
    wa = w97[0:32, :]
    wb = w97[32:64, :]
    wc = w97[64:96, :]
    wd = w97[96:97, :]
    c1 = _dot(p1[...], wa) + _dot(p2[...], wb) + _dot(p3[...], wc)
    c1 = c1 + p4[..., 0:1] * wd + c1b[...]
    c1 = jnp.maximum(c1, 0.0)
    r = c1.reshape(G, K, 16).reshape(G, K // 2, 2, 16).max(axis=2)
    acc = jnp.zeros((G, 11, 32), f32)
    for w in range(5):
        acc = acc + lax.dot_general(
            r[:, w:w + 11, :], w2[w], (((2,), (0,)), ((), ())),
            precision=_HI, preferred_element_type=f32)
    c2 = jnp.maximum(acc + c2b[...], 0.0)
    cflat[...] = c2.reshape(G, 352)


def _tc_head1(p1, p2, p3, p4, w97, c1b, w2, c2b):
    return pl.pallas_call(
        _head1_body,
        out_shape=jax.ShapeDtypeStruct((G, 352), f32),
    )(p1, p2, p3, p4, w97, c1b, w2, c2b)


def _head2_body(cf, w, b, o):
    o[...] = (_dot(cf[...], w[0]) + b[0])[None]


def _tc_head2(cflat, linwr, linb):
    nv = linwr.shape[2]
    return pl.pallas_call(
        _head2_body,
        grid=(5,),
        in_specs=[
            pl.BlockSpec((G, 352), lambda i: (0, 0)),
            pl.BlockSpec((1, 352, nv), lambda i: (i, 0, 0)),
            pl.BlockSpec((1, 1, nv), lambda i: (i, 0, 0)),
        ],
        out_specs=pl.BlockSpec((1, G, nv), lambda i: (i, 0, 0)),
        out_shape=jax.ShapeDtypeStruct((5, G, nv), f32),
    )(cflat, linwr, linb.reshape(5, 1, nv))


def kernel(x, edge_index, node_depth, batch, type_table, attr_table,
           depth_table, W1, b1, W2, b2, W3, b3, W4, b4,
           conv1w, conv1b, conv2w, conv2b, linW, linb):
    x0 = jnp.pad(x[:, 0].astype(i32), (0, P - N))
    x1 = jnp.pad(x[:, 1].astype(i32), (0, P - N))
    dep = jnp.pad(node_depth[:, 0].astype(i32), (0, P - N))
    src = edge_index[0].astype(i32)
    dst = edge_index[1].astype(i32)
    batch_p = jnp.pad(batch.astype(i32), (0, P - N), constant_values=BSENT)
    batchf2 = batch_p.astype(f32).reshape(NRT, RT)
    b01 = jnp.stack([batch_p.reshape(NRT, RT)[:, 0],
                     batch_p.reshape(NRT, RT)[:, RT - 1]], axis=1)
    zeros16 = jnp.zeros((P, HALF), f32)
    onesz2 = jnp.zeros((2, P, HALF), f32).at[0, :, 0].set(1.0)
    slots_init = jnp.full((SLOTS_PAD,), SENT, i32)
    b1r = b1.reshape(1, HID)
    b2r = b2.reshape(1, HID)
    b3r = b3.reshape(1, HID)
    b4r = b4.reshape(1, 1)
    w97 = jnp.transpose(conv1w[:, 0, :])
    c1br = conv1b.reshape(1, 16)
    w2t = jnp.transpose(conv2w, (2, 1, 0))
    c2br = conv2b.reshape(1, 1, 32)
    linwr = jnp.transpose(linW.reshape(5, 32, 11, 5000),
                          (0, 2, 1, 3)).reshape(5, 352, 5000)

    t1, a1, d1 = _tc_premul(type_table, attr_table, depth_table, W1)
    z1 = _sc_emb(x0, x1, dep, t1, a1, d1)
    deg2 = _sc_agg(src, dst, onesz2, zeros16)
    dinv, zs = _tc_prep1(deg2, z1)

    acc = _sc_agg(src, dst, zs, zeros16)
    h1, zs = _tc_epi(acc, zs, dinv, b1r, W2)
    acc = _sc_agg(src, dst, zs, zeros16)
    h2, zs = _tc_epi(acc, zs, dinv, b2r, W3)
    acc = _sc_agg(src, dst, zs, zeros16)
    h3, zs = _tc_epi(acc, zs, dinv, b3r, W4, last=True)
    acc = _sc_agg(src, dst, zs, zeros16)
    h4p = _tc_epi4(acc, zs, dinv, b4r)

    keys2 = h4p[:, 0].reshape(NRT, RT)
    st, en = _tc_ends(batchf2)
    se = jnp.concatenate([st.reshape(G, 1), en.reshape(G, 1)],
                         axis=1).astype(i32)
    dest2 = _tc_rank(se, b01, keys2, batchf2)
    slots = _sc_slots(dest2.reshape(P), slots_init)
    p1, p2, p3, p4 = _sc_pool(slots, h1, h2, h3, h4p)

    cflat = _tc_head1(p1, p2, p3, p4, w97, c1br, w2t, c2br)
    out = _tc_head2(cflat, linwr, linb)
    return tuple(out[idx] for idx in range(5))


# speedup vs baseline: 17.3820x; 1.1065x over previous
import functools

import jax
import jax.numpy as jnp
from jax import lax
from jax.experimental import pallas as pl
from jax.experimental.pallas import tpu as pltpu
from jax.experimental.pallas import tpu_sc as plsc

f32 = jnp.float32
i32 = jnp.int32

N = 100000
P = 100352
E = 1600000
G = 128
K = 30
HID = 32
HALF = 16
NB = P // 1024
RT = 256
NRT = P // RT
TPB = P // 32
EMB_B = 784
EC = 800
SLOTS = G * K
SLOTS_PAD = 3848
SENT = N
BSENT = 999

_HI = jax.lax.Precision.HIGHEST
_mesh = plsc.VectorSubcoreMesh(core_axis_name="c", subcore_axis_name="s")


def _dot(a, b):
    return jnp.dot(a, b, precision=_HI, preferred_element_type=f32)


@functools.partial(
    pl.kernel,
    out_type=jax.ShapeDtypeStruct((P, HID), f32),
    mesh=_mesh,
    compiler_params=pltpu.CompilerParams(use_tc_tiling_on_sc=False),
    scratch_types=[
        pltpu.VMEM((EMB_B,), i32),
        pltpu.VMEM((EMB_B,), i32),
        pltpu.VMEM((EMB_B,), i32),
        pltpu.VMEM((EMB_B, HID), f32),
        pltpu.VMEM((EMB_B, HID), f32),
        pltpu.VMEM((EMB_B, HID), f32),
        pltpu.SemaphoreType.DMA,
    ],
)
def _sc_emb(x0, x1, dep, t1, a1, d1, z1, i0, i1, i2, bt, ba, bd, sem):
    c = lax.axis_index("c")
    s = lax.axis_index("s")
    wid = s * 2 + c
    nb = wid * TPB
    for b in range(4):
        base = nb + b * EMB_B
        pltpu.sync_copy(x0.at[pl.ds(base, EMB_B)], i0)
        pltpu.sync_copy(x1.at[pl.ds(base, EMB_B)], i1)
        pltpu.sync_copy(dep.at[pl.ds(base, EMB_B)], i2)
        pltpu.async_copy(t1.at[i0], bt, sem).wait()
        pltpu.async_copy(a1.at[i1], ba, sem).wait()
        pltpu.async_copy(d1.at[i2], bd, sem).wait()

        def add(e, carry):
            for h in range(2):
                sl = pl.ds(h * HALF, HALF)
                bt[e, sl] = bt[e, sl] + ba[e, sl] + bd[e, sl]
            return carry

        lax.fori_loop(0, EMB_B, add, 0)
        pltpu.sync_copy(bt, z1.at[pl.ds(base, EMB_B)])


@functools.partial(
    pl.kernel,
    out_type=jax.ShapeDtypeStruct((2, P, HALF), f32),
    mesh=_mesh,
    compiler_params=pltpu.CompilerParams(use_tc_tiling_on_sc=False),
    scratch_types=[
        pltpu.VMEM((EC,), i32),
        pltpu.VMEM((EC,), i32),
        pltpu.VMEM((EC, HALF), f32),
        pltpu.VMEM((EC,), i32),
        pltpu.VMEM((EC,), i32),
        pltpu.VMEM((EC, HALF), f32),
        pltpu.VMEM_SHARED((P, HALF), f32),
        pltpu.SemaphoreType.DMA,
        pltpu.SemaphoreType.DMA,
        pltpu.SemaphoreType.DMA,
        pltpu.SemaphoreType.DMA,
    ],
)
def _sc_agg(src, dst, zs2, zeros16, acc_out,
            sbuf0, dbuf0, rows0, sbuf1, dbuf1, rows1, acc,
            gsem0, gsem1, ssem0, ssem1):
    c = lax.axis_index("c")
    s = lax.axis_index("s")
    pltpu.sync_copy(zeros16.at[pl.ds(s * TPB, TPB)], acc.at[pl.ds(s * TPB, TPB)])
    plsc.subcore_barrier()
    ebase = s * (E // 16)

    def pair(p, carry):
        e0 = ebase + p * (2 * EC)
        pltpu.sync_copy(src.at[pl.ds(e0, EC)], sbuf0)
        pltpu.sync_copy(dst.at[pl.ds(e0, EC)], dbuf0)
        g0 = pltpu.async_copy(zs2.at[c].at[sbuf0], rows0, gsem0)
        pltpu.sync_copy(src.at[pl.ds(e0 + EC, EC)], sbuf1)
        pltpu.sync_copy(dst.at[pl.ds(e0 + EC, EC)], dbuf1)
        g1 = pltpu.async_copy(zs2.at[c].at[sbuf1], rows1, gsem1)
        g0.wait()
        s0 = pltpu.async_copy(rows0, acc.at[dbuf0], ssem0, add=True)
        g1.wait()
        s1 = pltpu.async_copy(rows1, acc.at[dbuf1], ssem1, add=True)
        s0.wait()
        s1.wait()
        return carry

    npairs = (E // 16) // (2 * EC)
    lax.fori_loop(0, npairs, pair, 0)
    ntail = (E // 16) - npairs * 2 * EC
    if ntail:
        assert ntail == EC
        e0 = ebase + npairs * 2 * EC
        pltpu.sync_copy(src.at[pl.ds(e0, EC)], sbuf0)
        pltpu.sync_copy(dst.at[pl.ds(e0, EC)], dbuf0)
        pltpu.async_copy(zs2.at[c].at[sbuf0], rows0, gsem0).wait()
        pltpu.sync_copy(rows0, acc.at[dbuf0], add=True)
    plsc.subcore_barrier()
    pltpu.sync_copy(acc.at[pl.ds(s * TPB, TPB)], acc_out.at[c, pl.ds(s * TPB, TPB)])


@functools.partial(
    pl.kernel,
    out_type=jax.ShapeDtypeStruct((SLOTS,), i32),
    mesh=_mesh,
    compiler_params=pltpu.CompilerParams(use_tc_tiling_on_sc=False,
                                         needs_layout_passes=False),
    scratch_types=[
        pltpu.VMEM((1024,), i32),
        pltpu.VMEM((SLOTS_PAD,), i32),
    ],
)
def _sc_slots(dest, init, slots_out, dbuf, slots):
    c = lax.axis_index("c")
    s = lax.axis_index("s")

    @pl.when(jnp.logical_and(c == 0, s == 0))
    def _():
        pltpu.sync_copy(init, slots)

        def blk(b, carry):
            pltpu.sync_copy(dest.at[pl.ds(b * 1024, 1024)], dbuf)

            def inner(k, c2):
                iv = dbuf[pl.ds(k * 16, 16)]
                vals = lax.iota(i32, 16) + (b * 1024 + k * 16)
                plsc.store_scatter(slots, [iv], vals)
                return c2

            lax.fori_loop(0, 64, inner, 0)
            return carry

        lax.fori_loop(0, P // 1024, blk, 0)
        pltpu.sync_copy(slots.at[pl.ds(0, SLOTS)], slots_out)


@functools.partial(
    pl.kernel,
    out_type=[
        jax.ShapeDtypeStruct((SLOTS, HID), f32),
        jax.ShapeDtypeStruct((SLOTS, HID), f32),
        jax.ShapeDtypeStruct((SLOTS, HID), f32),
        jax.ShapeDtypeStruct((SLOTS, HALF), f32),
    ],
    mesh=_mesh,
    compiler_params=pltpu.CompilerParams(use_tc_tiling_on_sc=False),
    scratch_types=[
        pltpu.VMEM((SLOTS // 32,), i32),
        pltpu.VMEM((SLOTS // 32, HID), f32),
        pltpu.VMEM((SLOTS // 32, HALF), f32),
        pltpu.SemaphoreType.DMA,
    ],
)
def _sc_pool(slots, h1, h2, h3, h4p, p1, p2, p3, p4, ibuf, b32, b16, sem):
    c = lax.axis_index("c")
    s = lax.axis_index("s")
    wid = s * 2 + c
    nb = SLOTS // 32
    base = wid * nb
    pltpu.sync_copy(slots.at[pl.ds(base, nb)], ibuf)
    for href, pref in ((h1, p1), (h2, p2), (h3, p3)):
        pltpu.async_copy(href.at[ibuf], b32, sem).wait()
        pltpu.sync_copy(b32, pref.at[pl.ds(base, nb)])
    pltpu.async_copy(h4p.at[ibuf], b16, sem).wait()
    pltpu.sync_copy(b16, p4.at[pl.ds(base, nb)])


def _premul_body(tt, at_, dt, w1, o1, o2, o3):
    o1[...] = _dot(tt[...], w1[...])
    o2[...] = _dot(at_[...], w1[...])
    o3[...] = _dot(dt[...], w1[...])


def _tc_premul(type_table, attr_table, depth_table, w1):
    nt, na, nd = type_table.shape[0], attr_table.shape[0], depth_table.shape[0]
    return pl.pallas_call(
        _premul_body,
        out_shape=[
            jax.ShapeDtypeStruct((nt, HID), f32),
            jax.ShapeDtypeStruct((na, HID), f32),
            jax.ShapeDtypeStruct((nd, HID), f32),
        ],
    )(type_table, attr_table, depth_table, w1)


def _prep1_body(deg2, z1, dinv, zs2):
    d = deg2[0, :, 0:1] + deg2[1, :, 0:1] + 1.0
    di = lax.rsqrt(d)
    zs = di * z1[...]
    dinv[...] = di
    zs2[...] = jnp.stack([zs[:, 0:HALF], zs[:, HALF:HID]], axis=0)


def _tc_prep1(deg2, z1):
    return pl.pallas_call(
        _prep1_body,
        grid=(NB,),
        in_specs=[
            pl.BlockSpec((2, 1024, HALF), lambda i: (0, i, 0)),
            pl.BlockSpec((1024, HID), lambda i: (i, 0)),
        ],
        out_specs=[
            pl.BlockSpec((1024, 1), lambda i: (i, 0)),
            pl.BlockSpec((2, 1024, HALF), lambda i: (0, i, 0)),
        ],
        out_shape=[
            jax.ShapeDtypeStruct((P, 1), f32),
            jax.ShapeDtypeStruct((2, P, HALF), f32),
        ],
    )(deg2, z1)


def _epi_body(acc2, zs2, dinv, bvec, wnext, h_out, zsn2, *, last):
    pid = pl.program_id(0)
    di = dinv[...]
    z = jnp.concatenate([acc2[0] + zs2[0], acc2[1] + zs2[1]], axis=1)
    h = jnp.tanh(di * z + bvec[...])
    rid = pid * 1024 + lax.broadcasted_iota(i32, (1024, 1), 0)
    h = jnp.where(rid < N, h, 0.0)
    h_out[...] = h
    zs = di * _dot(h, wnext[...])
    if last:
        onehot = (lax.broadcasted_iota(i32, (1, HALF), 1) == 0).astype(f32)
        zs = zs * onehot
        zsn2[...] = jnp.stack([zs, zs], axis=0)
    else:
        zsn2[...] = jnp.stack([zs[:, 0:HALF], zs[:, HALF:HID]], axis=0)


def _tc_epi(acc2, zs2, dinv, bvec, wnext, last=False):
    wn = wnext.shape[1]
    return pl.pallas_call(
        functools.partial(_epi_body, last=last),
        grid=(NB,),
        in_specs=[
            pl.BlockSpec((2, 1024, HALF), lambda i: (0, i, 0)),
            pl.BlockSpec((2, 1024, HALF), lambda i: (0, i, 0)),
            pl.BlockSpec((1024, 1), lambda i: (i, 0)),
            pl.BlockSpec((1, HID), lambda i: (0, 0)),
            pl.BlockSpec((HID, wn), lambda i: (0, 0)),
        ],
        out_specs=[
            pl.BlockSpec((1024, HID), lambda i: (i, 0)),
            pl.BlockSpec((2, 1024, HALF), lambda i: (0, i, 0)),
        ],
        out_shape=[
            jax.ShapeDtypeStruct((P, HID), f32),
            jax.ShapeDtypeStruct((2, P, HALF), f32),
        ],
    )(acc2, zs2, dinv, bvec, wnext)


def _epi4_body(acc2, zs2, dinv, bvec, h4p):
    pid = pl.program_id(0)
    a = acc2[0, :, 0:1] + zs2[0, :, 0:1]
    h = jnp.tanh(dinv[...] * a + bvec[...])
    rid = pid * 1024 + lax.broadcasted_iota(i32, (1024, 1), 0)
    h = jnp.where(rid < N, h, 0.0)
    onehot = (lax.broadcasted_iota(i32, (1, HALF), 1) == 0).astype(f32)
    h4p[...] = h * onehot


def _tc_epi4(acc2, zs2, dinv, bvec):
    return pl.pallas_call(
        _epi4_body,
        grid=(NB,),
        in_specs=[
            pl.BlockSpec((2, 1024, HALF), lambda i: (0, i, 0)),
            pl.BlockSpec((2, 1024, HALF), lambda i: (0, i, 0)),
            pl.BlockSpec((1024, 1), lambda i: (i, 0)),
            pl.BlockSpec((1, 1), lambda i: (0, 0)),
        ],
        out_specs=pl.BlockSpec((1024, HALF), lambda i: (i, 0)),
        out_shape=jax.ShapeDtypeStruct((P, HALF), f32),
    )(acc2, zs2, dinv, bvec)


def _ends_body(bb, st, en):
    pid = pl.program_id(0)

    @pl.when(pid == 0)
    def _():
        st[...] = jnp.zeros_like(st[...])
        en[...] = jnp.zeros_like(en[...])

    bmat = jnp.transpose(jnp.broadcast_to(bb[0], (RT, RT)))[:, 0:G]
    g = lax.broadcasted_iota(i32, (RT, G), 1).astype(f32)
    st[...] += jnp.sum((bmat < g).astype(f32), axis=0, keepdims=True)
    en[...] += jnp.sum((bmat <= g).astype(f32), axis=0, keepdims=True)


def _tc_ends(batchf2):
    return pl.pallas_call(
        _ends_body,
        grid=(NRT,),
        in_specs=[pl.BlockSpec((1, 1, RT), lambda i: (i, 0, 0))],
        out_specs=[
            pl.BlockSpec((1, G), lambda i: (0, 0)),
            pl.BlockSpec((1, G), lambda i: (0, 0)),
        ],
        out_shape=[
            jax.ShapeDtypeStruct((1, G), f32),
            jax.ShapeDtypeStruct((1, G), f32),
        ],
    )(batchf2.reshape(NRT, 1, RT))


def _rank_body(se, b01, keys, bat, dest):
    t = pl.program_id(0)
    b0 = b01[t, 0]
    b1 = b01[t, 1]
    jlo = se[b0, 0]
    jhi = se[b1, 1]
    c0 = jlo // RT
    c1 = (jhi + RT - 1) // RT
    ki = jnp.broadcast_to(keys[pl.ds(t, 1), :], (RT, RT))
    bi = jnp.broadcast_to(bat[pl.ds(t, 1), :], (RT, RT))
    ii = lax.broadcasted_iota(i32, (RT, RT), 1) + t * RT

    def chunk(cc, cnt):
        kj = jnp.transpose(jnp.broadcast_to(keys[pl.ds(cc, 1), :], (RT, RT)))
        bj = jnp.transpose(jnp.broadcast_to(bat[pl.ds(cc, 1), :], (RT, RT)))
        jj = lax.broadcasted_iota(i32, (RT, RT), 0) + cc * RT
        better = (kj > ki) | ((kj == ki) & (jj < ii))
        m = (bj == bi) & better
        return cnt + jnp.sum(m.astype(f32), axis=0)

    cnt = lax.fori_loop(c0, c1, chunk, jnp.zeros((RT,), f32))
    rank = cnt.astype(i32).reshape(1, RT)
    bi_i = bat[pl.ds(t, 1), :].astype(i32)
    valid = (bi_i < G) & (rank < K)
    dest[...] = jnp.where(valid, bi_i * K + rank, SLOTS)[None]


def _tc_rank(se, b01, keys2, batchf2):
    return pl.pallas_call(
        _rank_body,
        grid=(NRT,),
        in_specs=[
            pl.BlockSpec(memory_space=pltpu.SMEM),
            pl.BlockSpec(memory_space=pltpu.SMEM),
            pl.BlockSpec((NRT, RT), lambda i: (0, 0)),
            pl.BlockSpec((NRT, RT), lambda i: (0, 0)),
        ],
        out_specs=pl.BlockSpec((1, 1, RT), lambda i: (i, 0, 0)),
        out_shape=jax.ShapeDtypeStruct((NRT, 1, RT), i32),
    )(se, b01, keys2, batchf2)


def _head1_body(p1, p2, p3, p4, w97, c1b, w2, c2b, cflat):
    wa = w97[0:32, :]
    wb = w97[32:64, :]
    wc = w97[64:96, :]
    wd = w97[96:97, :]
    c1 = _dot(p1[...], wa) + _dot(p2[...], wb) + _dot(p3[...], wc)
    c1 = c1 + p4[..., 0:1] * wd + c1b[...]
    c1 = jnp.maximum(c1, 0.0)
    r = c1.reshape(G, K, 16).reshape(G, K // 2, 2, 16).max(axis=2)
    acc = jnp.zeros((G, 11, 32), f32)
    for w in range(5):
        acc = acc + lax.dot_general(
            r[:, w:w + 11, :], w2[w], (((2,), (0,)), ((), ())),
            precision=_HI, preferred_element_type=f32)
    c2 = jnp.maximum(acc + c2b[...], 0.0)
    cflat[...] = c2.reshape(G, 352)


def _tc_head1(p1, p2, p3, p4, w97, c1b, w2, c2b):
    return pl.pallas_call(
        _head1_body,
        out_shape=jax.ShapeDtypeStruct((G, 352), f32),
    )(p1, p2, p3, p4, w97, c1b, w2, c2b)


def _head2_body(cf, w, b, o):
    o[...] = (_dot(cf[...], w[0]) + b[0])[None]


def _tc_head2(cflat, linwr, linb):
    nv = linwr.shape[2]
    return pl.pallas_call(
        _head2_body,
        grid=(5,),
        in_specs=[
            pl.BlockSpec((G, 352), lambda i: (0, 0)),
            pl.BlockSpec((1, 352, nv), lambda i: (i, 0, 0)),
            pl.BlockSpec((1, 1, nv), lambda i: (i, 0, 0)),
        ],
        out_specs=pl.BlockSpec((1, G, nv), lambda i: (i, 0, 0)),
        out_shape=jax.ShapeDtypeStruct((5, G, nv), f32),
    )(cflat, linwr, linb.reshape(5, 1, nv))


def kernel(x, edge_index, node_depth, batch, type_table, attr_table,
           depth_table, W1, b1, W2, b2, W3, b3, W4, b4,
           conv1w, conv1b, conv2w, conv2b, linW, linb):
    x0 = jnp.pad(x[:, 0].astype(i32), (0, P - N))
    x1 = jnp.pad(x[:, 1].astype(i32), (0, P - N))
    dep = jnp.pad(node_depth[:, 0].astype(i32), (0, P - N))
    src = edge_index[0].astype(i32)
    dst = edge_index[1].astype(i32)
    batch_p = jnp.pad(batch.astype(i32), (0, P - N), constant_values=BSENT)
    batchf2 = batch_p.astype(f32).reshape(NRT, RT)
    b01 = jnp.stack([batch_p.reshape(NRT, RT)[:, 0],
                     batch_p.reshape(NRT, RT)[:, RT - 1]], axis=1)
    zeros16 = jnp.zeros((P, HALF), f32)
    onesz2 = jnp.zeros((2, P, HALF), f32).at[0, :, 0].set(1.0)
    slots_init = jnp.full((SLOTS_PAD,), SENT, i32)
    b1r = b1.reshape(1, HID)
    b2r = b2.reshape(1, HID)
    b3r = b3.reshape(1, HID)
    b4r = b4.reshape(1, 1)
    w97 = jnp.transpose(conv1w[:, 0, :])
    c1br = conv1b.reshape(1, 16)
    w2t = jnp.transpose(conv2w, (2, 1, 0))
    c2br = conv2b.reshape(1, 1, 32)
    linwr = jnp.transpose(linW.reshape(5, 32, 11, 5000),
                          (0, 2, 1, 3)).reshape(5, 352, 5000)

    t1, a1, d1 = _tc_premul(type_table, attr_table, depth_table, W1)
    z1 = _sc_emb(x0, x1, dep, t1, a1, d1)
    deg2 = _sc_agg(src, dst, onesz2, zeros16)
    dinv, zs = _tc_prep1(deg2, z1)

    acc = _sc_agg(src, dst, zs, zeros16)
    h1, zs = _tc_epi(acc, zs, dinv, b1r, W2)
    acc = _sc_agg(src, dst, zs, zeros16)
    h2, zs = _tc_epi(acc, zs, dinv, b2r, W3)
    acc = _sc_agg(src, dst, zs, zeros16)
    h3, zs = _tc_epi(acc, zs, dinv, b3r, W4, last=True)
    acc = _sc_agg(src, dst, zs, zeros16)
    h4p = _tc_epi4(acc, zs, dinv, b4r)

    keys2 = h4p[:, 0].reshape(NRT, RT)
    st, en = _tc_ends(batchf2)
    se = jnp.concatenate([st.reshape(G, 1), en.reshape(G, 1)],
                         axis=1).astype(i32)
    dest2 = _tc_rank(se, b01, keys2, batchf2)
    slots = _sc_slots(dest2.reshape(P), slots_init)
    p1, p2, p3, p4 = _sc_pool(slots, h1, h2, h3, h4p)

    cflat = _tc_head1(p1, p2, p3, p4, w97, c1br, w2t, c2br)
    out = _tc_head2(cflat, linwr, linb)
    return tuple(out[idx] for idx in range(5))


# speedup vs baseline: 25.3951x; 1.4610x over previous
import functools

import jax
import jax.numpy as jnp
from jax import lax
from jax.experimental import pallas as pl
from jax.experimental.pallas import tpu as pltpu
from jax.experimental.pallas import tpu_sc as plsc

f32 = jnp.float32
i32 = jnp.int32

N = 100000
P = 100352
E = 1600000
G = 128
K = 30
HID = 32
HALF = 16
NB = P // 1024
RT = 256
NRT = P // RT
TPB = P // 32
EMB_B = 784
EC = 800
SLOTS = G * K
SLOTS_PAD = 3848
PR = P // 8
PB = PR // 8
SENT = N
BSENT = 999

_HI = jax.lax.Precision.HIGHEST
_mesh = plsc.VectorSubcoreMesh(core_axis_name="c", subcore_axis_name="s")


def _dot(a, b):
    return jnp.dot(a, b, precision=_HI, preferred_element_type=f32)


@functools.partial(
    pl.kernel,
    out_type=jax.ShapeDtypeStruct((P, HID), f32),
    mesh=_mesh,
    compiler_params=pltpu.CompilerParams(use_tc_tiling_on_sc=False),
    scratch_types=[
        pltpu.VMEM((EMB_B,), i32),
        pltpu.VMEM((EMB_B,), i32),
        pltpu.VMEM((EMB_B,), i32),
        pltpu.VMEM((EMB_B, HID), f32),
        pltpu.VMEM((EMB_B, HID), f32),
        pltpu.VMEM((EMB_B, HID), f32),
        pltpu.SemaphoreType.DMA,
    ],
)
def _sc_emb(x0, x1, dep, t1, a1, d1, z1, i0, i1, i2, bt, ba, bd, sem):
    c = lax.axis_index("c")
    s = lax.axis_index("s")
    wid = s * 2 + c
    nb = wid * TPB
    for b in range(4):
        base = nb + b * EMB_B
        pltpu.sync_copy(x0.at[pl.ds(base, EMB_B)], i0)
        pltpu.sync_copy(x1.at[pl.ds(base, EMB_B)], i1)
        pltpu.sync_copy(dep.at[pl.ds(base, EMB_B)], i2)
        pltpu.async_copy(t1.at[i0], bt, sem).wait()
        pltpu.async_copy(a1.at[i1], ba, sem).wait()
        pltpu.async_copy(d1.at[i2], bd, sem).wait()

        def add(e, carry):
            for h in range(2):
                sl = pl.ds(h * HALF, HALF)
                bt[e, sl] = bt[e, sl] + ba[e, sl] + bd[e, sl]
            return carry

        lax.fori_loop(0, EMB_B, add, 0)
        pltpu.sync_copy(bt, z1.at[pl.ds(base, EMB_B)])


@functools.partial(
    pl.kernel,
    out_type=jax.ShapeDtypeStruct((2, P, HALF), f32),
    mesh=_mesh,
    compiler_params=pltpu.CompilerParams(use_tc_tiling_on_sc=False),
    scratch_types=[
        pltpu.VMEM((EC,), i32),
        pltpu.VMEM((EC,), i32),
        pltpu.VMEM((EC, HALF), f32),
        pltpu.VMEM((EC,), i32),
        pltpu.VMEM((EC,), i32),
        pltpu.VMEM((EC, HALF), f32),
        pltpu.VMEM_SHARED((P, HALF), f32),
        pltpu.SemaphoreType.DMA,
        pltpu.SemaphoreType.DMA,
        pltpu.SemaphoreType.DMA,
        pltpu.SemaphoreType.DMA,
    ],
)
def _sc_agg(src, dst, zs2, zeros16, acc_out,
            sbuf0, dbuf0, rows0, sbuf1, dbuf1, rows1, acc,
            gsem0, gsem1, ssem0, ssem1):
    c = lax.axis_index("c")
    s = lax.axis_index("s")
    pltpu.sync_copy(zeros16, acc.at[pl.ds(s * TPB, TPB)])
    plsc.subcore_barrier()
    ebase = s * (E // 16)

    def pair(p, carry):
        e0 = ebase + p * (2 * EC)
        pltpu.sync_copy(src.at[pl.ds(e0, EC)], sbuf0)
        pltpu.sync_copy(dst.at[pl.ds(e0, EC)], dbuf0)
        g0 = pltpu.async_copy(zs2.at[c].at[sbuf0], rows0, gsem0)
        pltpu.sync_copy(src.at[pl.ds(e0 + EC, EC)], sbuf1)
        pltpu.sync_copy(dst.at[pl.ds(e0 + EC, EC)], dbuf1)
        g1 = pltpu.async_copy(zs2.at[c].at[sbuf1], rows1, gsem1)
        g0.wait()
        s0 = pltpu.async_copy(rows0, acc.at[dbuf0], ssem0, add=True)
        g1.wait()
        s1 = pltpu.async_copy(rows1, acc.at[dbuf1], ssem1, add=True)
        s0.wait()
        s1.wait()
        return carry

    npairs = (E // 16) // (2 * EC)
    lax.fori_loop(0, npairs, pair, 0)
    ntail = (E // 16) - npairs * 2 * EC
    if ntail:
        assert ntail == EC
        e0 = ebase + npairs * 2 * EC
        pltpu.sync_copy(src.at[pl.ds(e0, EC)], sbuf0)
        pltpu.sync_copy(dst.at[pl.ds(e0, EC)], dbuf0)
        pltpu.async_copy(zs2.at[c].at[sbuf0], rows0, gsem0).wait()
        pltpu.sync_copy(rows0, acc.at[dbuf0], add=True)
    plsc.subcore_barrier()
    pltpu.sync_copy(acc.at[pl.ds(s * TPB, TPB)], acc_out.at[c, pl.ds(s * TPB, TPB)])


@functools.partial(
    pl.kernel,
    out_type=jax.ShapeDtypeStruct((SLOTS,), i32),
    mesh=_mesh,
    compiler_params=pltpu.CompilerParams(use_tc_tiling_on_sc=False,
                                         needs_layout_passes=False),
    scratch_types=[
        pltpu.VMEM((1024,), i32),
        pltpu.VMEM((SLOTS_PAD,), i32),
    ],
)
def _sc_slots(dest, init, slots_out, dbuf, slots):
    c = lax.axis_index("c")
    s = lax.axis_index("s")

    @pl.when(jnp.logical_and(c == 0, s == 0))
    def _():
        pltpu.sync_copy(init, slots)

        def blk(b, carry):
            pltpu.sync_copy(dest.at[pl.ds(b * 1024, 1024)], dbuf)

            def inner(k, c2):
                iv = dbuf[pl.ds(k * 16, 16)]
                vals = lax.iota(i32, 16) + (b * 1024 + k * 16)
                plsc.store_scatter(slots, [iv], vals)
                return c2

            lax.fori_loop(0, 64, inner, 0)
            return carry

        lax.fori_loop(0, P // 1024, blk, 0)
        pltpu.sync_copy(slots.at[pl.ds(0, SLOTS)], slots_out)


@functools.partial(
    pl.kernel,
    out_type=[jax.ShapeDtypeStruct((SLOTS, HALF), f32) for _ in range(7)],
    mesh=_mesh,
    compiler_params=pltpu.CompilerParams(use_tc_tiling_on_sc=False),
    scratch_types=[
        pltpu.VMEM((SLOTS // 32,), i32),
        pltpu.VMEM((SLOTS // 32, HALF), f32),
        pltpu.SemaphoreType.DMA,
    ],
)
def _sc_pool(slots, h1, h2, h3, h4p, p1l, p1h, p2l, p2h, p3l, p3h, p4,
             ibuf, b16, sem):
    c = lax.axis_index("c")
    s = lax.axis_index("s")
    wid = s * 2 + c
    nb = SLOTS // 32
    base = wid * nb
    pltpu.sync_copy(slots.at[pl.ds(base, nb)], ibuf)
    for arr, plo, phi in ((h1, p1l, p1h), (h2, p2l, p2h), (h3, p3l, p3h)):
        for hh, pref in ((0, plo), (1, phi)):
            pltpu.async_copy(arr.at[hh].at[ibuf], b16, sem).wait()
            pltpu.sync_copy(b16, pref.at[pl.ds(base, nb)])
    pltpu.async_copy(h4p.at[ibuf], b16, sem).wait()
    pltpu.sync_copy(b16, p4.at[pl.ds(base, nb)])


def _premul_body(tt, at_, dt, w1, o1, o2, o3):
    o1[...] = _dot(tt[...], w1[...])
    o2[...] = _dot(at_[...], w1[...])
    o3[...] = _dot(dt[...], w1[...])


def _tc_premul(type_table, attr_table, depth_table, w1):
    nt, na, nd = type_table.shape[0], attr_table.shape[0], depth_table.shape[0]
    return pl.pallas_call(
        _premul_body,
        out_shape=[
            jax.ShapeDtypeStruct((nt, HID), f32),
            jax.ShapeDtypeStruct((na, HID), f32),
            jax.ShapeDtypeStruct((nd, HID), f32),
        ],
    )(type_table, attr_table, depth_table, w1)


def _prep1_body(deg2, z1p, mb, sel, dinvp, zs2):
    d = _dot(deg2[0] + deg2[1], mb[...]) + 1.0
    di = lax.rsqrt(d)
    zcat = _dot(z1p[...], sel[...])
    dinvp[...] = di
    zs2[...] = jnp.stack([di * zcat[:, 0:128], di * zcat[:, 128:256]], axis=0)


def _tc_prep1(deg2p, z1p, mb, sel):
    return pl.pallas_call(
        _prep1_body,
        grid=(8,),
        in_specs=[
            pl.BlockSpec((2, PB, 128), lambda i: (0, i, 0)),
            pl.BlockSpec((PB, 256), lambda i: (i, 0)),
            pl.BlockSpec((128, 128), lambda i: (0, 0)),
            pl.BlockSpec((256, 256), lambda i: (0, 0)),
        ],
        out_specs=[
            pl.BlockSpec((PB, 128), lambda i: (i, 0)),
            pl.BlockSpec((2, PB, 128), lambda i: (0, i, 0)),
        ],
        out_shape=[
            jax.ShapeDtypeStruct((PR, 128), f32),
            jax.ShapeDtypeStruct((2, PR, 128), f32),
        ],
    )(deg2p, z1p, mb, sel)


def _epi_body(acc2, zs2, dinvp, btile, bdw, h_out, zsn2, *, last):
    pid = pl.program_id(0)
    di = dinvp[...]
    hcat = jnp.concatenate([di * (acc2[0] + zs2[0]), di * (acc2[1] + zs2[1])],
                           axis=1)
    hcat = jnp.tanh(hcat + btile[...])
    r = pid * PB + lax.broadcasted_iota(i32, (PB, 256), 0)
    kcol = (lax.broadcasted_iota(i32, (PB, 256), 1) % 128) // 16
    hcat = jnp.where(r * 8 + kcol < N, hcat, 0.0)
    h_out[...] = jnp.stack([hcat[:, 0:128], hcat[:, 128:256]], axis=0)
    zcat = _dot(hcat, bdw[...])
    if last:
        zs = di * zcat
        zsn2[...] = jnp.stack([zs, zs], axis=0)
    else:
        zsn2[...] = jnp.stack([di * zcat[:, 0:128], di * zcat[:, 128:256]],
                              axis=0)


def _tc_epi(acc2p, zs2p, dinvp, btile, bdw, last=False):
    bk = bdw.shape[1]
    return pl.pallas_call(
        functools.partial(_epi_body, last=last),
        grid=(8,),
        in_specs=[
            pl.BlockSpec((2, PB, 128), lambda i: (0, i, 0)),
            pl.BlockSpec((2, PB, 128), lambda i: (0, i, 0)),
            pl.BlockSpec((PB, 128), lambda i: (i, 0)),
            pl.BlockSpec((1, 256), lambda i: (0, 0)),
            pl.BlockSpec((256, bk), lambda i: (0, 0)),
        ],
        out_specs=[
            pl.BlockSpec((2, PB, 128), lambda i: (0, i, 0)),
            pl.BlockSpec((2, PB, 128), lambda i: (0, i, 0)),
        ],
        out_shape=[
            jax.ShapeDtypeStruct((2, PR, 128), f32),
            jax.ShapeDtypeStruct((2, PR, 128), f32),
        ],
    )(acc2p, zs2p, dinvp, btile, bdw)


def _epi4_body(acc2, zs2, dinvp, b4tile, h4p):
    pid = pl.program_id(0)
    h = jnp.tanh(dinvp[...] * (acc2[0] + zs2[0]) + b4tile[...])
    r = pid * PB + lax.broadcasted_iota(i32, (PB, 128), 0)
    kcol = lax.broadcasted_iota(i32, (PB, 128), 1) // 16
    h4p[...] = jnp.where(r * 8 + kcol < N, h, 0.0)


def _tc_epi4(acc2p, zs2p, dinvp, b4tile):
    return pl.pallas_call(
        _epi4_body,
        grid=(8,),
        in_specs=[
            pl.BlockSpec((2, PB, 128), lambda i: (0, i, 0)),
            pl.BlockSpec((2, PB, 128), lambda i: (0, i, 0)),
            pl.BlockSpec((PB, 128), lambda i: (i, 0)),
            pl.BlockSpec((1, 128), lambda i: (0, 0)),
        ],
        out_specs=pl.BlockSpec((PB, 128), lambda i: (i, 0)),
        out_shape=jax.ShapeDtypeStruct((PR, 128), f32),
    )(acc2p, zs2p, dinvp, b4tile)


def _ends_body(bb, st, en):
    pid = pl.program_id(0)

    @pl.when(pid == 0)
    def _():
        st[...] = jnp.zeros_like(st[...])
        en[...] = jnp.zeros_like(en[...])

    bmat = jnp.transpose(jnp.broadcast_to(bb[0], (RT, RT)))[:, 0:G]
    g = lax.broadcasted_iota(i32, (RT, G), 1).astype(f32)
    st[...] += jnp.sum((bmat < g).astype(f32), axis=0, keepdims=True)
    en[...] += jnp.sum((bmat <= g).astype(f32), axis=0, keepdims=True)


def _tc_ends(batchf2):
    return pl.pallas_call(
        _ends_body,
        grid=(NRT,),
        in_specs=[pl.BlockSpec((1, 1, RT), lambda i: (i, 0, 0))],
        out_specs=[
            pl.BlockSpec((1, G), lambda i: (0, 0)),
            pl.BlockSpec((1, G), lambda i: (0, 0)),
        ],
        out_shape=[
            jax.ShapeDtypeStruct((1, G), f32),
            jax.ShapeDtypeStruct((1, G), f32),
        ],
    )(batchf2.reshape(NRT, 1, RT))


def _rank_body(se, b01, keys, bat, dest):
    t = pl.program_id(0)
    b0 = b01[t, 0]
    b1 = b01[t, 1]
    jlo = se[b0, 0]
    jhi = se[b1, 1]
    c0 = jlo // RT
    c1 = (jhi + RT - 1) // RT
    ki = jnp.broadcast_to(keys[pl.ds(t, 1), :], (RT, RT))
    bi = jnp.broadcast_to(bat[pl.ds(t, 1), :], (RT, RT))
    ii = lax.broadcasted_iota(i32, (RT, RT), 1) + t * RT

    def chunk(cc, cnt):
        kj = jnp.transpose(jnp.broadcast_to(keys[pl.ds(cc, 1), :], (RT, RT)))
        bj = jnp.transpose(jnp.broadcast_to(bat[pl.ds(cc, 1), :], (RT, RT)))
        jj = lax.broadcasted_iota(i32, (RT, RT), 0) + cc * RT
        better = (kj > ki) | ((kj == ki) & (jj < ii))
        m = (bj == bi) & better
        return cnt + jnp.sum(m.astype(f32), axis=0)

    cnt = lax.fori_loop(c0, c1, chunk, jnp.zeros((RT,), f32))
    rank = cnt.astype(i32).reshape(1, RT)
    bi_i = bat[pl.ds(t, 1), :].astype(i32)
    valid = (bi_i < G) & (rank < K)
    dest[...] = jnp.where(valid, bi_i * K + rank, SLOTS)[None]


def _tc_rank(se, b01, keys2, batchf2):
    return pl.pallas_call(
        _rank_body,
        grid=(NRT,),
        in_specs=[
            pl.BlockSpec(memory_space=pltpu.SMEM),
            pl.BlockSpec(memory_space=pltpu.SMEM),
            pl.BlockSpec((NRT, RT), lambda i: (0, 0)),
            pl.BlockSpec((NRT, RT), lambda i: (0, 0)),
        ],
        out_specs=pl.BlockSpec((1, 1, RT), lambda i: (i, 0, 0)),
        out_shape=jax.ShapeDtypeStruct((NRT, 1, RT), i32),
    )(se, b01, keys2, batchf2)


def _head1_body(p1l, p1h, p2l, p2h, p3l, p3h, p4, w97, c1b, w2, c2b, cflat):
    c1 = (_dot(p1l[...], w97[0:16, :]) + _dot(p1h[...], w97[16:32, :])
          + _dot(p2l[...], w97[32:48, :]) + _dot(p2h[...], w97[48:64, :])
          + _dot(p3l[...], w97[64:80, :]) + _dot(p3h[...], w97[80:96, :]))
    c1 = c1 + p4[..., 0:1] * w97[96:97, :] + c1b[...]
    c1 = jnp.maximum(c1, 0.0)
    r = c1.reshape(G, K, 16).reshape(G, K // 2, 2, 16).max(axis=2)
    acc = jnp.zeros((G, 11, 32), f32)
    for w in range(5):
        acc = acc + lax.dot_general(
            r[:, w:w + 11, :], w2[w], (((2,), (0,)), ((), ())),
            precision=_HI, preferred_element_type=f32)
    c2 = jnp.maximum(acc + c2b[...], 0.0)
    cflat[...] = jnp.transpose(c2, (0, 2, 1)).reshape(G, 352)


def _tc_head1(pooled, w97, c1b, w2, c2b):
    return pl.pallas_call(
        _head1_body,
        out_shape=jax.ShapeDtypeStruct((G, 352), f32),
    )(*pooled, w97, c1b, w2, c2b)


def _head2_body(cf, w, b, o):
    o[...] = (_dot(cf[...], w[0]) + b[0])[None]


def _tc_head2(cflat, linwr, linb):
    nv = linwr.shape[2]
    return pl.pallas_call(
        _head2_body,
        grid=(5,),
        in_specs=[
            pl.BlockSpec((G, 352), lambda i: (0, 0)),
            pl.BlockSpec((1, 352, nv), lambda i: (i, 0, 0)),
            pl.BlockSpec((1, 1, nv), lambda i: (i, 0, 0)),
        ],
        out_specs=pl.BlockSpec((1, G, nv), lambda i: (i, 0, 0)),
        out_shape=jax.ShapeDtypeStruct((5, G, nv), f32),
    )(cflat, linwr, linb.reshape(5, 1, nv))


def kernel(x, edge_index, node_depth, batch, type_table, attr_table,
           depth_table, W1, b1, W2, b2, W3, b3, W4, b4,
           conv1w, conv1b, conv2w, conv2b, linW, linb):
    x0 = jnp.pad(x[:, 0].astype(i32), (0, P - N))
    x1 = jnp.pad(x[:, 1].astype(i32), (0, P - N))
    dep = jnp.pad(node_depth[:, 0].astype(i32), (0, P - N))
    src = edge_index[0].astype(i32)
    dst = edge_index[1].astype(i32)
    batch_p = jnp.pad(batch.astype(i32), (0, P - N), constant_values=BSENT)
    batchf2 = batch_p.astype(f32).reshape(NRT, RT)
    b01 = jnp.stack([batch_p.reshape(NRT, RT)[:, 0],
                     batch_p.reshape(NRT, RT)[:, RT - 1]], axis=1)
    zeros_small = jnp.zeros((TPB, HALF), f32)
    onesz2 = jnp.zeros((2, P, HALF), f32).at[0, :, 0].set(1.0)
    slots_init = jnp.full((SLOTS_PAD,), SENT, i32)
    w97 = jnp.transpose(conv1w[:, 0, :])
    c1br = conv1b.reshape(1, 16)
    w2t = jnp.transpose(conv2w, (2, 1, 0))
    c2br = conv2b.reshape(1, 1, 32)
    lanes = jnp.arange(128)
    mb = (lanes[:, None] == (lanes[None, :] // 16) * 16).astype(f32)
    cols = jnp.arange(256)
    half = cols // 128
    kk = (cols % 128) // 16
    ch = cols % 16
    src_col = kk * 32 + half * 16 + ch
    sel = (jnp.arange(256)[:, None] == src_col[None, :]).astype(f32)
    e8 = jnp.eye(8, dtype=f32)

    def _bdw(w):
        return jnp.concatenate([
            jnp.concatenate([jnp.kron(e8, w[0:16, 0:16]),
                             jnp.kron(e8, w[0:16, 16:32])], axis=1),
            jnp.concatenate([jnp.kron(e8, w[16:32, 0:16]),
                             jnp.kron(e8, w[16:32, 16:32])], axis=1),
        ], axis=0)

    w4l = jnp.zeros((16, 16), f32).at[:, 0:1].set(W4[0:16])
    w4h = jnp.zeros((16, 16), f32).at[:, 0:1].set(W4[16:32])
    bdw4 = jnp.concatenate([jnp.kron(e8, w4l), jnp.kron(e8, w4h)], axis=0)

    def _btile(b):
        return jnp.concatenate([jnp.tile(b[0:16], 8),
                                jnp.tile(b[16:32], 8)]).reshape(1, 256)

    b4tile = jnp.tile(jnp.concatenate([b4, jnp.zeros((15,), f32)]),
                      8).reshape(1, 128)

    t1, a1, d1 = _tc_premul(type_table, attr_table, depth_table, W1)
    z1 = _sc_emb(x0, x1, dep, t1, a1, d1)
    deg2 = _sc_agg(src, dst, onesz2, zeros_small)
    dinvp, zsp = _tc_prep1(deg2.reshape(2, PR, 128), z1.reshape(PR, 256),
                           mb, sel)

    acc = _sc_agg(src, dst, zsp.reshape(2, P, HALF), zeros_small)
    h1p, zsp = _tc_epi(acc.reshape(2, PR, 128), zsp, dinvp, _btile(b1),
                       _bdw(W2))
    acc = _sc_agg(src, dst, zsp.reshape(2, P, HALF), zeros_small)
    h2p, zsp = _tc_epi(acc.reshape(2, PR, 128), zsp, dinvp, _btile(b2),
                       _bdw(W3))
    acc = _sc_agg(src, dst, zsp.reshape(2, P, HALF), zeros_small)
    h3p, zsp = _tc_epi(acc.reshape(2, PR, 128), zsp, dinvp, _btile(b3),
                       bdw4, last=True)
    acc = _sc_agg(src, dst, zsp.reshape(2, P, HALF), zeros_small)
    h4pp = _tc_epi4(acc.reshape(2, PR, 128), zsp, dinvp, b4tile)

    h4p = h4pp.reshape(P, HALF)
    keys2 = h4p[:, 0].reshape(NRT, RT)
    st, en = _tc_ends(batchf2)
    se = jnp.concatenate([st.reshape(G, 1), en.reshape(G, 1)],
                         axis=1).astype(i32)
    dest2 = _tc_rank(se, b01, keys2, batchf2)
    slots = _sc_slots(dest2.reshape(P), slots_init)
    pooled = _sc_pool(slots, h1p.reshape(2, P, HALF), h2p.reshape(2, P, HALF),
                      h3p.reshape(2, P, HALF), h4p)

    cflat = _tc_head1(pooled, w97, c1br, w2t, c2br)
    out = _tc_head2(cflat, linW, linb)
    return tuple(out[idx] for idx in range(5))
